# baseline (speedup 1.0000x reference)
import functools

import jax
import jax.numpy as jnp
from jax import lax
from jax.experimental import pallas as pl
from jax.experimental.pallas import tpu as pltpu
from jax.experimental.pallas import tpu_sc as plsc

_H = 16
_W = 16
_T = 4
_NLEV = 4
_NPTS = 4
_D = 256
_LEN = _T * _H * _W
_SHIFTS = (-1 - _H, -_H, 1 - _H, -1, 0, 1, -1 + _H, _H, 1 + _H)
_MAXG = (_H - 1) * (_W - 1)
_IDX_PAD = 128
_NWORK = 32


def _shift_rows(x, delta):
    if delta == 0:
        return x
    n, d = x.shape
    z = jnp.zeros((abs(delta), d), x.dtype)
    if delta > 0:
        return jnp.concatenate([x[delta:], z], axis=0)
    return jnp.concatenate([z, x[:n + delta]], axis=0)


def _corr_topk_body(q_ref, v_ref, idx_ref):
    q = q_ref[0]
    v = v_ref[0]
    row = lax.broadcasted_iota(jnp.int32, (_LEN, 1), 0)
    yy = (row // _W) % _H
    xx = row % _W
    corr = jnp.zeros((_LEN, _LEN), jnp.float32)
    for di in (-1, 0, 1):
        for dj in (-1, 0, 1):
            delta = di * _W + dj
            mask = ((yy + di >= 0) & (yy + di < _H)
                    & (xx + dj >= 0) & (xx + dj < _W)).astype(jnp.float32)
            qs = _shift_rows(q, delta) * mask
            vs = _shift_rows(v, delta) * mask
            corr = corr + lax.dot_general(
                qs, vs, (((1,), (1,)), ((), ())),
                preferred_element_type=jnp.float32)

    lane = lax.broadcasted_iota(jnp.int32, (_LEN, _IDX_PAD), 1)
    iot = lax.broadcasted_iota(jnp.int32, (_LEN, _D), 1)
    acc = jnp.zeros((_LEN, _IDX_PAD), jnp.int32)
    for lev in range(_NLEV):
        x = (corr[:, lev * _D:(lev + 1) * _D] * (1.0 / _D)) * 10.0
        mx = jnp.max(x, axis=1, keepdims=True)
        e = jnp.exp(x - mx)
        p = e / jnp.sum(e, axis=1, keepdims=True)
        for j in range(_NPTS):
            pm = jnp.max(p, axis=1, keepdims=True)
            am = jnp.min(jnp.where(p == pm, iot, _D), axis=1, keepdims=True)
            acc = jnp.where(lane == lev * _NPTS + j, am, acc)
            p = jnp.where(iot == am, -1.0, p)
    idx_ref[0] = acc


def _corr_topk(query, input_flatten):
    n = query.shape[0]
    return pl.pallas_call(
        _corr_topk_body,
        grid=(n,),
        in_specs=[
            pl.BlockSpec((1, _LEN, _D), lambda i: (i, 0, 0)),
            pl.BlockSpec((1, _LEN, _D), lambda i: (i, 0, 0)),
        ],
        out_specs=pl.BlockSpec((1, _LEN, _IDX_PAD), lambda i: (i, 0, 0)),
        out_shape=jax.ShapeDtypeStruct((n, _LEN, _IDX_PAD), jnp.int32),
    )(query, input_flatten)


def _expand_sc(idx_flat, nq):
    per_w = nq // _NWORK
    n_groups = per_w // 4
    mesh = plsc.VectorSubcoreMesh(core_axis_name="c", subcore_axis_name="s")

    @functools.partial(
        pl.kernel,
        mesh=mesh,
        compiler_params=pltpu.CompilerParams(needs_layout_passes=False),
        out_type=[
            jax.ShapeDtypeStruct((nq, _LEN), jnp.float32),
            jax.ShapeDtypeStruct((nq, 288), jnp.float32),
        ],
        scratch_types=[
            pltpu.VMEM((per_w * _IDX_PAD,), jnp.int32),
            pltpu.VMEM((per_w, _LEN), jnp.float32),
            pltpu.VMEM((per_w, 288), jnp.float32),
        ],
    )
    def sc_kernel(idx_hbm, w_hbm, loc_hbm, idx_v, w_v, loc_v):
        cid = lax.axis_index("c")
        sid = lax.axis_index("s")
        wid = sid * 2 + cid
        qoff = wid * per_w
        pltpu.sync_copy(idx_hbm.at[pl.ds(qoff * _IDX_PAD, per_w * _IDX_PAD)],
                        idx_v)
        lane = lax.iota(jnp.int32, 16)
        qs = lane >> 2
        lev = lane & 3
        col0 = qs * _IDX_PAD + lev * _NPTS
        loc0 = lev * 72
        wcol0 = lev * _D
        tapw = jnp.full((16,), 1.0 / 576.0, jnp.float32)
        zero16 = jnp.zeros((16,), jnp.float32)

        def zero_blk(r, carry):
            for u in range(_LEN // 16):
                w_v[r, pl.ds(u * 16, 16)] = zero16
            return carry

        lax.fori_loop(0, per_w, zero_blk, 0)

        @plsc.parallel_loop(0, n_groups)
        def group(g):
            qrow = g * 4 + qs
            for pt in range(_NPTS):
                m0 = plsc.load_gather(idx_v, [g * 512 + col0 + pt])
                for si, s in enumerate(_SHIFTS):
                    m = jnp.clip(m0 + s, 0, _MAXG)
                    r_ = m >> 4
                    c_ = m & 15
                    jj = si * _NPTS + pt
                    plsc.store_scatter(loc_v, [qrow, loc0 + 2 * jj],
                                       r_.astype(jnp.float32) * 0.0625)
                    plsc.store_scatter(loc_v, [qrow, loc0 + 2 * jj + 1],
                                       c_.astype(jnp.float32) * 0.0625)
                    for dr in (-1, 0):
                        for dc in (-1, 0):
                            rr = r_ + dr
                            cc = c_ + dc
                            msk = (rr >= 0) & (cc >= 0)
                            pix = jnp.maximum((cc << 4) + rr, 0)
                            plsc.addupdate_scatter(
                                w_v, [qrow, wcol0 + pix], tapw, mask=msk)
        pltpu.sync_copy(w_v, w_hbm.at[pl.ds(qoff, per_w)])
        pltpu.sync_copy(loc_v, loc_hbm.at[pl.ds(qoff, per_w)])

    return sc_kernel(idx_flat)


def _apply_body(w_ref, v_ref, wo_ref, b_ref, o_ref):
    t = lax.dot_general(w_ref[...], v_ref[0], (((1,), (0,)), ((), ())),
                        preferred_element_type=jnp.float32)
    o = lax.dot_general(t, wo_ref[...], (((1,), (1,)), ((), ())),
                        preferred_element_type=jnp.float32)
    o_ref[0] = o + b_ref[...]


def _apply(weights, input_flatten, w_out, b_out):
    n = input_flatten.shape[0]
    return pl.pallas_call(
        _apply_body,
        grid=(n,),
        in_specs=[
            pl.BlockSpec((_LEN, _LEN), lambda i: (i, 0)),
            pl.BlockSpec((1, _LEN, _D), lambda i: (i, 0, 0)),
            pl.BlockSpec((_D, _D), lambda i: (0, 0)),
            pl.BlockSpec((1, _D), lambda i: (0, 0)),
        ],
        out_specs=pl.BlockSpec((1, _LEN, _D), lambda i: (i, 0, 0)),
        out_shape=jax.ShapeDtypeStruct((n, _LEN, _D), jnp.float32),
    )(weights, input_flatten, w_out, b_out.reshape(1, _D))


def kernel(query, reference_points, input_flatten, input_spatial_shapes,
           input_level_start_index, W_out, b_out):
    n, len_q, _ = query.shape
    idx = _corr_topk(query, input_flatten)
    w2d, loc2d = _expand_sc(idx.reshape(-1), n * len_q)
    out = _apply(w2d, input_flatten, W_out, b_out)
    sampling_locations = loc2d.reshape(n, len_q, 1, _NLEV, 36, 2)
    return (out, sampling_locations)


# speedup vs baseline: 1.0306x; 1.0306x over previous
import functools

import jax
import jax.numpy as jnp
from jax import lax
from jax.experimental import pallas as pl
from jax.experimental.pallas import tpu as pltpu
from jax.experimental.pallas import tpu_sc as plsc

_H = 16
_W = 16
_T = 4
_NLEV = 4
_NPTS = 4
_D = 256
_LEN = _T * _H * _W
_SHIFTS = (-1 - _H, -_H, 1 - _H, -1, 0, 1, -1 + _H, _H, 1 + _H)
_MAXG = (_H - 1) * (_W - 1)
_IDX_PAD = 128
_NWORK = 32


def _shift_rows(x, delta):
    if delta == 0:
        return x
    n, d = x.shape
    z = jnp.zeros((abs(delta), d), x.dtype)
    if delta > 0:
        return jnp.concatenate([x[delta:], z], axis=0)
    return jnp.concatenate([z, x[:n + delta]], axis=0)


def _corr_topk_body(q_ref, v_ref, idx_ref):
    q = q_ref[0]
    v = v_ref[0]
    row = lax.broadcasted_iota(jnp.int32, (_LEN, 1), 0)
    yy = (row // _W) % _H
    xx = row % _W
    corr = jnp.zeros((_LEN, _LEN), jnp.float32)
    for di in (-1, 0, 1):
        for dj in (-1, 0, 1):
            delta = di * _W + dj
            mask = ((yy + di >= 0) & (yy + di < _H)
                    & (xx + dj >= 0) & (xx + dj < _W)).astype(jnp.float32)
            qs = _shift_rows(q, delta) * mask
            vs = _shift_rows(v, delta) * mask
            corr = corr + lax.dot_general(
                qs, vs, (((1,), (1,)), ((), ())),
                preferred_element_type=jnp.float32)

    lane = lax.broadcasted_iota(jnp.int32, (_LEN, _IDX_PAD), 1)
    iot = lax.broadcasted_iota(jnp.int32, (_LEN, _D), 1)
    acc = jnp.zeros((_LEN, _IDX_PAD), jnp.int32)
    for lev in range(_NLEV):
        x = (corr[:, lev * _D:(lev + 1) * _D] * (1.0 / _D)) * 10.0
        mx = jnp.max(x, axis=1, keepdims=True)
        e = jnp.exp(x - mx)
        p = e / jnp.sum(e, axis=1, keepdims=True)
        for j in range(_NPTS):
            pm = jnp.max(p, axis=1, keepdims=True)
            am = jnp.min(jnp.where(p == pm, iot, _D), axis=1, keepdims=True)
            acc = jnp.where(lane == lev * _NPTS + j, am, acc)
            p = jnp.where(iot == am, -1.0, p)
    idx_ref[0] = acc


def _corr_topk(query, input_flatten):
    n = query.shape[0]
    return pl.pallas_call(
        _corr_topk_body,
        grid=(n,),
        in_specs=[
            pl.BlockSpec((1, _LEN, _D), lambda i: (i, 0, 0)),
            pl.BlockSpec((1, _LEN, _D), lambda i: (i, 0, 0)),
        ],
        out_specs=pl.BlockSpec((1, _LEN, _IDX_PAD), lambda i: (i, 0, 0)),
        out_shape=jax.ShapeDtypeStruct((n, _LEN, _IDX_PAD), jnp.int32),
    )(query, input_flatten)


def _expand_sc(idx_flat, nq):
    per_w = nq // _NWORK
    n_groups = per_w // 4
    mesh = plsc.VectorSubcoreMesh(core_axis_name="c", subcore_axis_name="s")

    @functools.partial(
        pl.kernel,
        mesh=mesh,
        compiler_params=pltpu.CompilerParams(needs_layout_passes=False),
        out_type=[
            jax.ShapeDtypeStruct((nq, _LEN), jnp.float32),
            jax.ShapeDtypeStruct((nq, 288), jnp.float32),
        ],
        scratch_types=[
            pltpu.VMEM((per_w * _IDX_PAD,), jnp.int32),
            pltpu.VMEM((per_w, _LEN), jnp.float32),
            pltpu.VMEM((per_w, 288), jnp.float32),
        ],
    )
    def sc_kernel(idx_hbm, w_hbm, loc_hbm, idx_v, w_v, loc_v):
        cid = lax.axis_index("c")
        sid = lax.axis_index("s")
        wid = sid * 2 + cid
        qoff = wid * per_w
        pltpu.sync_copy(idx_hbm.at[pl.ds(qoff * _IDX_PAD, per_w * _IDX_PAD)],
                        idx_v)
        lane = lax.iota(jnp.int32, 16)
        qs = lane >> 2
        lev = lane & 3
        col0 = qs * _IDX_PAD + lev * _NPTS
        loc0 = lev * 72
        wcol0 = lev * _D
        base01 = wcol0 - 16
        base10 = wcol0 - 1
        base11 = wcol0 - 17
        tapw = jnp.full((16,), 1.0 / 576.0, jnp.float32)
        zero16 = jnp.zeros((16,), jnp.float32)

        def zero_blk(r, carry):
            for u in range(_LEN // 16):
                w_v[r, pl.ds(u * 16, 16)] = zero16
            return carry

        lax.fori_loop(0, per_w, zero_blk, 0)

        @plsc.parallel_loop(0, n_groups)
        def group(g):
            qrow = g * 4 + qs
            for pt in range(_NPTS):
                m0 = plsc.load_gather(idx_v, [g * 512 + col0 + pt])
                for si, s in enumerate(_SHIFTS):
                    m = jnp.clip(m0 + s, 0, _MAXG)
                    r_ = m >> 4
                    c_ = m & 15
                    jj = si * _NPTS + pt
                    plsc.store_scatter(loc_v, [qrow, loc0 + 2 * jj],
                                       r_.astype(jnp.float32) * 0.0625)
                    plsc.store_scatter(loc_v, [qrow, loc0 + 2 * jj + 1],
                                       c_.astype(jnp.float32) * 0.0625)
                    s1 = (c_ << 4) + r_
                    mr = r_ >= 1
                    mc = c_ >= 1
                    plsc.addupdate_scatter(w_v, [qrow, wcol0 + s1], tapw)
                    plsc.addupdate_scatter(w_v, [qrow, base01 + s1], tapw,
                                           mask=mc)
                    plsc.addupdate_scatter(w_v, [qrow, base10 + s1], tapw,
                                           mask=mr)
                    plsc.addupdate_scatter(w_v, [qrow, base11 + s1], tapw,
                                           mask=mr & mc)
        pltpu.sync_copy(w_v, w_hbm.at[pl.ds(qoff, per_w)])
        pltpu.sync_copy(loc_v, loc_hbm.at[pl.ds(qoff, per_w)])

    return sc_kernel(idx_flat)


def _apply_body(w_ref, v_ref, wo_ref, b_ref, o_ref):
    t = lax.dot_general(w_ref[...], v_ref[0], (((1,), (0,)), ((), ())),
                        preferred_element_type=jnp.float32)
    o = lax.dot_general(t, wo_ref[...], (((1,), (1,)), ((), ())),
                        preferred_element_type=jnp.float32)
    o_ref[0] = o + b_ref[...]


def _apply(weights, input_flatten, w_out, b_out):
    n = input_flatten.shape[0]
    return pl.pallas_call(
        _apply_body,
        grid=(n,),
        in_specs=[
            pl.BlockSpec((_LEN, _LEN), lambda i: (i, 0)),
            pl.BlockSpec((1, _LEN, _D), lambda i: (i, 0, 0)),
            pl.BlockSpec((_D, _D), lambda i: (0, 0)),
            pl.BlockSpec((1, _D), lambda i: (0, 0)),
        ],
        out_specs=pl.BlockSpec((1, _LEN, _D), lambda i: (i, 0, 0)),
        out_shape=jax.ShapeDtypeStruct((n, _LEN, _D), jnp.float32),
    )(weights, input_flatten, w_out, b_out.reshape(1, _D))


def kernel(query, reference_points, input_flatten, input_spatial_shapes,
           input_level_start_index, W_out, b_out):
    n, len_q, _ = query.shape
    idx = _corr_topk(query, input_flatten)
    w2d, loc2d = _expand_sc(idx.reshape(-1), n * len_q)
    out = _apply(w2d, input_flatten, W_out, b_out)
    sampling_locations = loc2d.reshape(n, len_q, 1, _NLEV, 36, 2)
    return (out, sampling_locations)
